# baseline (speedup 1.0000x reference)
import functools

import jax
import jax.numpy as jnp
from jax import lax
from jax.experimental import pallas as pl
from jax.experimental.pallas import tpu as pltpu
from jax.experimental.pallas import tpu_sc as plsc

_BATCH = 16384
_DIM = 64
_LANES = 16
_RPB = 8
_NBLK = 1000000 // _RPB

_info = plsc.get_sparse_core_info()
_NC, _NS = _info.num_cores, _info.num_subcores
_NW = _NC * _NS
_BPW = _BATCH // _NW
_K = _LANES
_NB = _BPW // _K


def _body(z_hbm, idx_hbm, tbl_hbm, out_hbm,
          idx_v, blk_v, z_v, out_v, bsems, zsems, osems):
    wid = lax.axis_index("s") * _NC + lax.axis_index("c")
    base = wid * _BPW
    pltpu.sync_copy(idx_hbm.at[pl.ds(base, _BPW)], idx_v)

    def fire(b, slot):
        lbl16 = idx_v[pl.ds(b * _K, _K)]
        for l in range(_K):
            o = lax.shift_right_logical(lbl16[l], 3)
            pltpu.async_copy(tbl_hbm.at[o], blk_v.at[slot, l], bsems.at[slot])
        pltpu.async_copy(z_hbm.at[pl.ds(base + b * _K, _K)], z_v.at[slot],
                         zsems.at[slot])

    def compute(b, slot):
        pltpu.make_async_copy(tbl_hbm.at[pl.ds(0, _K)],
                              blk_v.at[slot], bsems.at[slot]).wait()
        pltpu.make_async_copy(z_hbm.at[pl.ds(0, _K)], z_v.at[slot],
                              zsems.at[slot]).wait()
        lbl16 = idx_v[pl.ds(b * _K, _K)]
        for l in range(_K):
            rr = lax.bitwise_and(lbl16[l], 7)
            for c in range(_DIM // _LANES):
                s = pl.ds(c * _LANES, _LANES)
                out_v[slot, l, s] = blk_v[slot, l, rr, s] * z_v[slot, l, s]
        pltpu.async_copy(out_v.at[slot],
                         out_hbm.at[pl.ds(base + b * _K, _K)], osems.at[slot])

    def wait_out(slot):
        pltpu.make_async_copy(out_v.at[slot],
                              out_hbm.at[pl.ds(0, _K)], osems.at[slot]).wait()

    fire(0, 0)

    def pair(p):
        b0 = p * 2
        fire(b0 + 1, 1)
        compute(b0, 0)
        @pl.when(b0 + 2 < _NB)
        def _():
            wait_out(0)
            fire(b0 + 2, 0)
        compute(b0 + 1, 1)
        @pl.when(b0 + 2 < _NB)
        def _():
            wait_out(1)

    pl.loop(0, _NB // 2)(pair)
    wait_out(0)
    wait_out(1)


@functools.partial(jax.jit, donate_argnums=())
def kernel(z, label, embedding_table):
    mesh = plsc.VectorSubcoreMesh(core_axis_name="c", subcore_axis_name="s")
    tbl3 = embedding_table.reshape(_NBLK, _RPB, _DIM)
    k = functools.partial(
        pl.kernel,
        mesh=mesh,
        compiler_params=pltpu.CompilerParams(use_tc_tiling_on_sc=False),
        out_type=jax.ShapeDtypeStruct((_BATCH, _DIM), jnp.float32),
        scratch_types=[
            pltpu.VMEM((_BPW,), jnp.int32),
            pltpu.VMEM((2, _K, _RPB, _DIM), jnp.float32),
            pltpu.VMEM((2, _K, _DIM), jnp.float32),
            pltpu.VMEM((2, _K, _DIM), jnp.float32),
            pltpu.SemaphoreType.DMA((2,)),
            pltpu.SemaphoreType.DMA((2,)),
            pltpu.SemaphoreType.DMA((2,)),
        ],
    )(_body)
    return k(z, label.astype(jnp.int32), tbl3)


# speedup vs baseline: 1.0006x; 1.0006x over previous
import functools

import jax
import jax.numpy as jnp
from jax import lax
from jax.experimental import pallas as pl
from jax.experimental.pallas import tpu as pltpu
from jax.experimental.pallas import tpu_sc as plsc

_BATCH = 16384
_DIM = 64
_LANES = 16
_RPB = 8
_BW = 128
_LPB = 16
_NBLK = 1000000 // _LPB

_info = plsc.get_sparse_core_info()
_NC, _NS = _info.num_cores, _info.num_subcores
_NW = _NC * _NS
_BPW = _BATCH // _NW
_K = _LANES
_NB = _BPW // _K


def _body(z_hbm, idx_hbm, tbl_hbm, out_hbm,
          idx_v, blk_v, z_v, out_v, bsems, zsems, osems):
    wid = lax.axis_index("s") * _NC + lax.axis_index("c")
    base = wid * _BPW
    pltpu.sync_copy(idx_hbm.at[pl.ds(base, _BPW)], idx_v)

    def fire(b, slot):
        lbl16 = idx_v[pl.ds(b * _K, _K)]
        for l in range(_K):
            o = lax.shift_right_logical(lbl16[l], 4)
            pltpu.async_copy(tbl_hbm.at[o], blk_v.at[slot, l], bsems.at[slot])
        pltpu.async_copy(z_hbm.at[pl.ds(base + b * _K, _K)], z_v.at[slot],
                         zsems.at[slot])

    def compute(b, slot):
        pltpu.make_async_copy(tbl_hbm.at[pl.ds(0, _K)],
                              blk_v.at[slot], bsems.at[slot]).wait()
        pltpu.make_async_copy(z_hbm.at[pl.ds(0, _K)], z_v.at[slot],
                              zsems.at[slot]).wait()
        lbl16 = idx_v[pl.ds(b * _K, _K)]
        for l in range(_K):
            rr = lax.bitwise_and(lax.shift_right_logical(lbl16[l], 1), 7)
            h = lax.bitwise_and(lbl16[l], 1) * _DIM
            for c in range(_DIM // _LANES):
                s = pl.ds(c * _LANES, _LANES)
                out_v[slot, l, s] = (blk_v[slot, l, rr,
                                           pl.ds(h + c * _LANES, _LANES)]
                                     * z_v[slot, l, s])
        pltpu.async_copy(out_v.at[slot],
                         out_hbm.at[pl.ds(base + b * _K, _K)], osems.at[slot])

    def wait_out(slot):
        pltpu.make_async_copy(out_v.at[slot],
                              out_hbm.at[pl.ds(0, _K)], osems.at[slot]).wait()

    fire(0, 0)

    def pair(p):
        b0 = p * 2
        fire(b0 + 1, 1)
        compute(b0, 0)
        @pl.when(b0 + 2 < _NB)
        def _():
            wait_out(0)
            fire(b0 + 2, 0)
        compute(b0 + 1, 1)
        @pl.when(b0 + 2 < _NB)
        def _():
            wait_out(1)

    pl.loop(0, _NB // 2)(pair)
    wait_out(0)
    wait_out(1)


@functools.partial(jax.jit, donate_argnums=())
def kernel(z, label, embedding_table):
    mesh = plsc.VectorSubcoreMesh(core_axis_name="c", subcore_axis_name="s")
    tbl3 = embedding_table.reshape(_NBLK, _RPB, _BW)
    k = functools.partial(
        pl.kernel,
        mesh=mesh,
        compiler_params=pltpu.CompilerParams(use_tc_tiling_on_sc=True),
        out_type=jax.ShapeDtypeStruct((_BATCH, _DIM), jnp.float32),
        scratch_types=[
            pltpu.VMEM((_BPW,), jnp.int32),
            pltpu.VMEM((2, _K, _RPB, _BW), jnp.float32),
            pltpu.VMEM((2, _K, _DIM), jnp.float32),
            pltpu.VMEM((2, _K, _DIM), jnp.float32),
            pltpu.SemaphoreType.DMA((2,)),
            pltpu.SemaphoreType.DMA((2,)),
            pltpu.SemaphoreType.DMA((2,)),
        ],
    )(_body)
    return k(z, label.astype(jnp.int32), tbl3)


# speedup vs baseline: 2.4278x; 2.4265x over previous
import functools

import jax
import jax.numpy as jnp
from jax import lax
from jax.experimental import pallas as pl
from jax.experimental.pallas import tpu as pltpu
from jax.experimental.pallas import tpu_sc as plsc

_BATCH = 16384
_DIM = 64
_LANES = 16
_RPB = 8
_NBLK = 1000000 // _RPB

_info = plsc.get_sparse_core_info()
_NC, _NS = _info.num_cores, _info.num_subcores
_NW = _NC * _NS
_BPW = _BATCH // _NW
_K = _LANES
_NB = _BPW // _K
_NS_BUF = 4


def _body(z_hbm, idx_hbm, tbl_hbm, out_hbm,
          idx_v, blk_v, z_v, out_v, bsems, zsems, osems):
    wid = lax.axis_index("s") * _NC + lax.axis_index("c")
    base = wid * _BPW
    pltpu.sync_copy(idx_hbm.at[pl.ds(base, _BPW)], idx_v)

    def fire(b, slot):
        lbl16 = idx_v[pl.ds(b * _K, _K)]
        for l in range(_K):
            o = lax.shift_right_logical(lbl16[l], 3)
            pltpu.async_copy(tbl_hbm.at[o], blk_v.at[slot, l], bsems.at[slot])
        pltpu.async_copy(z_hbm.at[pl.ds(base + b * _K, _K)], z_v.at[slot],
                         zsems.at[slot])

    def compute(b, slot):
        pltpu.make_async_copy(tbl_hbm.at[pl.ds(0, _K)],
                              blk_v.at[slot], bsems.at[slot]).wait()
        pltpu.make_async_copy(z_hbm.at[pl.ds(0, _K)], z_v.at[slot],
                              zsems.at[slot]).wait()
        lbl16 = idx_v[pl.ds(b * _K, _K)]
        for l in range(_K):
            rr = lax.bitwise_and(lbl16[l], 7)
            for c in range(_DIM // _LANES):
                s = pl.ds(c * _LANES, _LANES)
                out_v[slot, l, s] = blk_v[slot, l, rr, s] * z_v[slot, l, s]
        pltpu.async_copy(out_v.at[slot],
                         out_hbm.at[pl.ds(base + b * _K, _K)], osems.at[slot])

    def wait_out(slot):
        pltpu.make_async_copy(out_v.at[slot],
                              out_hbm.at[pl.ds(0, _K)], osems.at[slot]).wait()

    for s in range(_NS_BUF - 1):
        fire(s, s)

    def quad(q):
        b0 = q * _NS_BUF
        for j in range(_NS_BUF):
            b = b0 + j
            nxt = b + (_NS_BUF - 1)
            s_nxt = (j + _NS_BUF - 1) % _NS_BUF
            @pl.when(jnp.logical_and(nxt < _NB, nxt >= _NS_BUF))
            def _():
                wait_out(s_nxt)
            @pl.when(nxt < _NB)
            def _():
                fire(nxt, s_nxt)
            compute(b, j)

    pl.loop(0, _NB // _NS_BUF)(quad)
    for s in range(_NS_BUF):
        wait_out(s)


@functools.partial(jax.jit, donate_argnums=())
def kernel(z, label, embedding_table):
    mesh = plsc.VectorSubcoreMesh(core_axis_name="c", subcore_axis_name="s")
    tbl3 = embedding_table.reshape(_NBLK, _RPB, _DIM)
    k = functools.partial(
        pl.kernel,
        mesh=mesh,
        compiler_params=pltpu.CompilerParams(use_tc_tiling_on_sc=True),
        out_type=jax.ShapeDtypeStruct((_BATCH, _DIM), jnp.float32),
        scratch_types=[
            pltpu.VMEM((_BPW,), jnp.int32),
            pltpu.VMEM((_NS_BUF, _K, _RPB, _DIM), jnp.float32),
            pltpu.VMEM((_NS_BUF, _K, _DIM), jnp.float32),
            pltpu.VMEM((_NS_BUF, _K, _DIM), jnp.float32),
            pltpu.SemaphoreType.DMA((_NS_BUF,)),
            pltpu.SemaphoreType.DMA((_NS_BUF,)),
            pltpu.SemaphoreType.DMA((_NS_BUF,)),
        ],
    )(_body)
    return k(z, label.astype(jnp.int32), tbl3)
